# Initial kernel scaffold; baseline (speedup 1.0000x reference)
import jax
import jax.numpy as jnp
from jax.experimental import pallas as pl


def kernel(x, edge_index, edge_attr, node_table, edge_W, edge_b, eps, W1, b1, g1, be1, W2, b2, g_out, b_out):
    raise NotImplementedError("write your pallas kernel here")


# speedup vs baseline: 3.1463x; 3.1463x over previous
import functools

import jax
import jax.numpy as jnp
from jax import lax
from jax.experimental import pallas as pl
from jax.experimental.pallas import tpu as pltpu
from jax.experimental.pallas import tpu_sc as plsc

N = 10000
E = 320000
D = 128
HID = 2 * D
EDGE_DIM = 7

NC = 2
NS = 16
NW = NC * NS
EPW = E // NW
CH = 80
NCH = EPW // CH
N_PAD = 10240
ZROWS = N_PAD // NS
OROWS = 624
OTAIL = N - NS * OROWS

BE = 3200


def _edge_body(ea_ref, w_ref, b_ref, m_ref, e_ref):
    y = jnp.dot(ea_ref[...], w_ref[...], preferred_element_type=jnp.float32)
    y = y + b_ref[...]
    m_ref[...] = jnp.maximum(y[:, :D], 0.0)
    e_ref[...] = y[:, D:]


def _edge_stage(interpret=False):
    return pl.pallas_call(
        _edge_body,
        grid=(E // BE,),
        in_specs=[
            pl.BlockSpec((BE, 8), lambda i: (i, 0)),
            pl.BlockSpec((8, 2 * D), lambda i: (0, 0)),
            pl.BlockSpec((1, 2 * D), lambda i: (0, 0)),
        ],
        out_specs=[
            pl.BlockSpec((BE, D), lambda i: (i, 0)),
            pl.BlockSpec((BE, D), lambda i: (i, 0)),
        ],
        out_shape=[
            jax.ShapeDtypeStruct((E, D), jnp.float32),
            jax.ShapeDtypeStruct((E, D), jnp.float32),
        ],
        interpret=interpret,
    )


def _make_sc_stage(with_gather, interpret=False):
    scratch = [
        pltpu.VMEM((CH,), jnp.int32),
        pltpu.VMEM((CH, D), jnp.float32),
        pltpu.VMEM_SHARED((N_PAD, D), jnp.float32),
    ]
    if with_gather:
        scratch += [
            pltpu.VMEM((CH,), jnp.int32),
            pltpu.VMEM((CH, D), jnp.float32),
        ]
    mesh = plsc.VectorSubcoreMesh(core_axis_name="c", subcore_axis_name="s")

    @functools.partial(
        pl.kernel,
        out_type=jax.ShapeDtypeStruct((NC, N, D), jnp.float32),
        mesh=mesh,
        scratch_types=scratch,
        interpret=interpret,
    )
    def sc_stage(*args):
        if with_gather:
            (emb_hbm, dst_hbm, z_hbm, src_hbm, h_hbm, out_hbm,
             dsti, ebuf, agg_sh, srci, rows) = args
        else:
            (emb_hbm, dst_hbm, z_hbm, out_hbm,
             dsti, ebuf, agg_sh) = args

        cid = lax.axis_index("c")
        sid = lax.axis_index("s")
        wid = cid * NS + sid

        pltpu.sync_copy(z_hbm, agg_sh.at[pl.ds(sid * ZROWS, ZROWS)])
        plsc.subcore_barrier()

        @pl.loop(0, NCH)
        def _(j):
            eb = wid * EPW + j * CH
            pltpu.sync_copy(dst_hbm.at[pl.ds(eb, CH)], dsti)
            pltpu.sync_copy(emb_hbm.at[pl.ds(eb, CH)], ebuf)
            if with_gather:
                pltpu.sync_copy(src_hbm.at[pl.ds(eb, CH)], srci)
                pltpu.sync_copy(h_hbm.at[srci], rows)

                @pl.loop(0, CH)
                def _(r):
                    for q in range(D // 16):
                        sl = pl.ds(q * 16, 16)
                        v = ebuf.at[r, sl][...] + rows.at[r, sl][...]
                        ebuf.at[r, sl][...] = jnp.maximum(v, 0.0)

            pltpu.sync_copy(ebuf, agg_sh.at[dsti], add=True)

        plsc.subcore_barrier()
        pltpu.sync_copy(
            agg_sh.at[pl.ds(sid * OROWS, OROWS)],
            out_hbm.at[cid, pl.ds(sid * OROWS, OROWS)],
        )

        @pl.when(sid == 0)
        def _():
            pltpu.sync_copy(
                agg_sh.at[pl.ds(NS * OROWS, OTAIL)],
                out_hbm.at[cid, pl.ds(NS * OROWS, OTAIL)],
            )

    return sc_stage


def _node_body(h_ref, a_ref, s_ref, w1_ref, b1_ref, g1_ref, be1_ref,
               w2_ref, b2_ref, go_ref, bo_ref, o_ref, *, final_relu):
    agg = a_ref[0] + a_ref[1]
    z = s_ref[0, 0] * h_ref[...] + agg
    t = jnp.dot(z, w1_ref[...], preferred_element_type=jnp.float32)
    t = t + b1_ref[...]
    m = jnp.mean(t, axis=0, keepdims=True)
    v = jnp.mean((t - m) ** 2, axis=0, keepdims=True)
    t = (t - m) * lax.rsqrt(v + 1e-5) * g1_ref[...] + be1_ref[...]
    t = jnp.maximum(t, 0.0)
    u = jnp.dot(t, w2_ref[...], preferred_element_type=jnp.float32)
    u = u + b2_ref[...]
    m2 = jnp.mean(u, axis=0, keepdims=True)
    v2 = jnp.mean((u - m2) ** 2, axis=0, keepdims=True)
    y = (u - m2) * lax.rsqrt(v2 + 1e-5) * go_ref[...] + bo_ref[...]
    if final_relu:
        y = jnp.maximum(y, 0.0)
    o_ref[...] = y


def _node_stage(final_relu, interpret=False):
    return pl.pallas_call(
        functools.partial(_node_body, final_relu=final_relu),
        out_shape=jax.ShapeDtypeStruct((N, D), jnp.float32),
        interpret=interpret,
    )


def kernel(x, edge_index, edge_attr, node_table, edge_W, edge_b, eps,
           W1, b1, g1, be1, W2, b2, g_out, b_out):
    f32 = jnp.float32
    src = edge_index[0]
    dst = edge_index[1]
    c = node_table[0].astype(f32)

    ea8 = jnp.pad(edge_attr.astype(f32), ((0, 0), (0, 8 - EDGE_DIM)))
    w8 = jnp.pad(
        jnp.concatenate([edge_W[0], edge_W[1]], axis=1).astype(f32),
        ((0, 8 - EDGE_DIM), (0, 0)),
    )
    bias = jnp.concatenate([edge_b[0] + c, edge_b[1]])[None].astype(f32)

    msg0, e1 = _edge_stage()(ea8, w8, bias)

    zeros = jnp.zeros((ZROWS, D), f32)
    agg0 = _make_sc_stage(with_gather=False)(msg0, dst, zeros)

    s0 = (1.0 + eps[0]).astype(f32).reshape(1, 1)
    h1 = _node_stage(final_relu=True)(
        c[None], agg0, s0, W1[0], b1[0][None], g1[0][None], be1[0][None],
        W2[0], b2[0][None], g_out[0][None], b_out[0][None])

    agg1 = _make_sc_stage(with_gather=True)(e1, dst, zeros, src, h1)

    s1 = (1.0 + eps[1]).astype(f32).reshape(1, 1)
    h2 = _node_stage(final_relu=False)(
        h1, agg1, s1, W1[1], b1[1][None], g1[1][None], be1[1][None],
        W2[1], b2[1][None], g_out[1][None], b_out[1][None])
    return h2


# speedup vs baseline: 4.3787x; 1.3917x over previous
import functools

import jax
import jax.numpy as jnp
from jax import lax
from jax.experimental import pallas as pl
from jax.experimental.pallas import tpu as pltpu
from jax.experimental.pallas import tpu_sc as plsc

N = 10000
E = 320000
D = 128
HID = 2 * D
EDGE_DIM = 7

NC = 2
NS = 16
NW = NC * NS
EPW = E // NW
CH = 80
NCH = EPW // CH
N_PAD = 10240
ZROWS = N_PAD // NS
OROWS = 624
OTAIL = N - NS * OROWS

BE = 3200


def _edge_body(ea_ref, w_ref, b_ref, o_ref, *, relu):
    y = jnp.dot(ea_ref[...], w_ref[...], preferred_element_type=jnp.float32)
    y = y + b_ref[...]
    if relu:
        y = jnp.maximum(y, 0.0)
    o_ref[...] = y


def _edge_stage(relu, interpret=False):
    return pl.pallas_call(
        functools.partial(_edge_body, relu=relu),
        grid=(E // BE,),
        in_specs=[
            pl.BlockSpec((BE, 8), lambda i: (i, 0)),
            pl.BlockSpec((8, D), lambda i: (0, 0)),
            pl.BlockSpec((1, D), lambda i: (0, 0)),
        ],
        out_specs=pl.BlockSpec((BE, D), lambda i: (i, 0)),
        out_shape=jax.ShapeDtypeStruct((E, D), jnp.float32),
        interpret=interpret,
    )


def _make_sc_stage(with_gather, interpret=False):
    scratch = [
        pltpu.VMEM((CH,), jnp.int32),
        pltpu.VMEM((CH,), jnp.int32),
        pltpu.VMEM((CH, D), jnp.float32),
        pltpu.VMEM((CH, D), jnp.float32),
        pltpu.VMEM_SHARED((N_PAD, D), jnp.float32),
        pltpu.SemaphoreType.DMA,
        pltpu.SemaphoreType.DMA,
        pltpu.SemaphoreType.DMA,
        pltpu.SemaphoreType.DMA,
    ]
    if with_gather:
        scratch += [
            pltpu.VMEM((CH,), jnp.int32),
            pltpu.VMEM((CH,), jnp.int32),
            pltpu.VMEM((CH, D), jnp.float32),
            pltpu.VMEM((CH, D), jnp.float32),
            pltpu.SemaphoreType.DMA,
            pltpu.SemaphoreType.DMA,
            pltpu.SemaphoreType.DMA,
            pltpu.SemaphoreType.DMA,
        ]
    mesh = plsc.VectorSubcoreMesh(core_axis_name="c", subcore_axis_name="s")

    @functools.partial(
        pl.kernel,
        out_type=jax.ShapeDtypeStruct((NC, N, D), jnp.float32),
        mesh=mesh,
        scratch_types=scratch,
        interpret=interpret,
    )
    def sc_stage(*args):
        if with_gather:
            (emb_hbm, dst_hbm, z_hbm, src_hbm, h_hbm, out_hbm,
             dsti0, dsti1, ebuf0, ebuf1, agg_sh, sd0, sd1, se0, se1,
             srci0, srci1, rows0, rows1, ss0, ss1, sg0, sg1) = args
            srci = (srci0, srci1)
            rows = (rows0, rows1)
            ssem = (ss0, ss1)
            gsem = (sg0, sg1)
        else:
            (emb_hbm, dst_hbm, z_hbm, out_hbm,
             dsti0, dsti1, ebuf0, ebuf1, agg_sh, sd0, sd1, se0, se1) = args
        dsti = (dsti0, dsti1)
        ebuf = (ebuf0, ebuf1)
        dsem = (sd0, sd1)
        esem = (se0, se1)

        cid = lax.axis_index("c")
        sid = lax.axis_index("s")
        wid = cid * NS + sid
        base = wid * EPW

        def start_loads(j, s):
            eb = base + j * CH
            pltpu.make_async_copy(
                dst_hbm.at[pl.ds(eb, CH)], dsti[s], dsem[s]).start()
            pltpu.make_async_copy(
                emb_hbm.at[pl.ds(eb, CH)], ebuf[s], esem[s]).start()
            if with_gather:
                pltpu.make_async_copy(
                    src_hbm.at[pl.ds(eb, CH)], srci[s], ssem[s]).start()

        def wait_loads(s):
            pltpu.make_async_copy(
                dst_hbm.at[pl.ds(0, CH)], dsti[s], dsem[s]).wait()
            pltpu.make_async_copy(
                emb_hbm.at[pl.ds(0, CH)], ebuf[s], esem[s]).wait()

        def start_gather(s):
            pltpu.make_async_copy(
                src_hbm.at[pl.ds(0, CH)], srci[s], ssem[s]).wait()
            pltpu.make_async_copy(h_hbm.at[srci[s]], rows[s], gsem[s]).start()

        def process(s):
            wait_loads(s)
            if with_gather:
                pltpu.make_async_copy(
                    h_hbm.at[srci[s]], rows[s], gsem[s]).wait()

                @pl.loop(0, CH, step=2)
                def _(r):
                    for rr in range(2):
                        for q in range(D // 16):
                            sl = pl.ds(q * 16, 16)
                            v = (ebuf[s].at[r + rr, sl][...]
                                 + rows[s].at[r + rr, sl][...])
                            ebuf[s].at[r + rr, sl][...] = jnp.maximum(v, 0.0)

                pltpu.sync_copy(ebuf[s], agg_sh.at[dsti[s]], add=True)
            else:
                pltpu.sync_copy(ebuf[s], agg_sh.at[dsti[s]], add=True)

        pltpu.sync_copy(z_hbm, agg_sh.at[pl.ds(sid * ZROWS, ZROWS)])
        plsc.subcore_barrier()

        start_loads(0, 0)
        start_loads(1, 1)
        if with_gather:
            start_gather(0)

        @pl.loop(0, (NCH - 1) // 2)
        def _(k):
            j = 2 * k
            process(0)
            start_loads(j + 2, 0)
            if with_gather:
                start_gather(1)
            process(1)

            @pl.when(j + 3 < NCH)
            def _():
                start_loads(j + 3, 1)

            if with_gather:
                start_gather(0)

        process(0)

        plsc.subcore_barrier()
        pltpu.sync_copy(
            agg_sh.at[pl.ds(sid * OROWS, OROWS)],
            out_hbm.at[cid, pl.ds(sid * OROWS, OROWS)],
        )

        @pl.when(sid == 0)
        def _():
            pltpu.sync_copy(
                agg_sh.at[pl.ds(NS * OROWS, OTAIL)],
                out_hbm.at[cid, pl.ds(NS * OROWS, OTAIL)],
            )

    return sc_stage


def _node_body(h_ref, a_ref, s_ref, w1_ref, b1_ref, g1_ref, be1_ref,
               w2_ref, b2_ref, go_ref, bo_ref, o_ref, *, final_relu):
    agg = a_ref[0] + a_ref[1]
    z = s_ref[0, 0] * h_ref[...] + agg
    t = jnp.dot(z, w1_ref[...], preferred_element_type=jnp.float32)
    t = t + b1_ref[...]
    m = jnp.mean(t, axis=0, keepdims=True)
    v = jnp.mean((t - m) ** 2, axis=0, keepdims=True)
    t = (t - m) * lax.rsqrt(v + 1e-5) * g1_ref[...] + be1_ref[...]
    t = jnp.maximum(t, 0.0)
    u = jnp.dot(t, w2_ref[...], preferred_element_type=jnp.float32)
    u = u + b2_ref[...]
    m2 = jnp.mean(u, axis=0, keepdims=True)
    v2 = jnp.mean((u - m2) ** 2, axis=0, keepdims=True)
    y = (u - m2) * lax.rsqrt(v2 + 1e-5) * go_ref[...] + bo_ref[...]
    if final_relu:
        y = jnp.maximum(y, 0.0)
    o_ref[...] = y


def _node_stage(final_relu, interpret=False):
    return pl.pallas_call(
        functools.partial(_node_body, final_relu=final_relu),
        out_shape=jax.ShapeDtypeStruct((N, D), jnp.float32),
        interpret=interpret,
    )


def kernel(x, edge_index, edge_attr, node_table, edge_W, edge_b, eps,
           W1, b1, g1, be1, W2, b2, g_out, b_out):
    f32 = jnp.float32
    src = edge_index[0]
    dst = edge_index[1]
    c = node_table[0].astype(f32)

    ea8 = jnp.pad(edge_attr.astype(f32), ((0, 0), (0, 8 - EDGE_DIM)))
    w0 = jnp.pad(edge_W[0].astype(f32), ((0, 8 - EDGE_DIM), (0, 0)))
    w1e = jnp.pad(edge_W[1].astype(f32), ((0, 8 - EDGE_DIM), (0, 0)))
    bias0 = (edge_b[0] + c)[None].astype(f32)
    bias1 = edge_b[1][None].astype(f32)

    msg0 = _edge_stage(relu=True)(ea8, w0, bias0)
    e1 = _edge_stage(relu=False)(ea8, w1e, bias1)

    zeros = jnp.zeros((ZROWS, D), f32)
    agg0 = _make_sc_stage(with_gather=False)(msg0, dst, zeros)

    s0 = (1.0 + eps[0]).astype(f32).reshape(1, 1)
    h1 = _node_stage(final_relu=True)(
        c[None], agg0, s0, W1[0], b1[0][None], g1[0][None], be1[0][None],
        W2[0], b2[0][None], g_out[0][None], b_out[0][None])

    agg1 = _make_sc_stage(with_gather=True)(e1, dst, zeros, src, h1)

    s1 = (1.0 + eps[1]).astype(f32).reshape(1, 1)
    h2 = _node_stage(final_relu=False)(
        h1, agg1, s1, W1[1], b1[1][None], g1[1][None], be1[1][None],
        W2[1], b2[1][None], g_out[1][None], b_out[1][None])
    return h2


# speedup vs baseline: 4.5161x; 1.0314x over previous
import functools

import jax
import jax.numpy as jnp
from jax import lax
from jax.experimental import pallas as pl
from jax.experimental.pallas import tpu as pltpu
from jax.experimental.pallas import tpu_sc as plsc

N = 10000
E = 320000
D = 128
HID = 2 * D
EDGE_DIM = 7

NC = 2
NS = 16
NW = NC * NS
EPW = E // NW
CH = 80
NCH = EPW // CH
N_PAD = 10240
ZROWS = N_PAD // NS
OROWS = 624
OTAIL = N - NS * OROWS

BE = 3200


def _edge_body(ea_ref, w_ref, b_ref, o_ref, *, relu):
    y = jnp.dot(ea_ref[...], w_ref[...], preferred_element_type=jnp.float32)
    y = y + b_ref[...]
    if relu:
        y = jnp.maximum(y, 0.0)
    o_ref[...] = y


def _edge_stage(relu, interpret=False):
    return pl.pallas_call(
        functools.partial(_edge_body, relu=relu),
        grid=(E // BE,),
        in_specs=[
            pl.BlockSpec((BE, EDGE_DIM), lambda i: (i, 0)),
            pl.BlockSpec((EDGE_DIM, D), lambda i: (0, 0)),
            pl.BlockSpec((1, D), lambda i: (0, 0)),
        ],
        out_specs=pl.BlockSpec((BE, D), lambda i: (i, 0)),
        out_shape=jax.ShapeDtypeStruct((E, D), jnp.float32),
        interpret=interpret,
    )


RING = 4
AHEAD = 2


def _make_sc_scatter(interpret=False):
    scratch = (
        [pltpu.VMEM((CH,), jnp.int32) for _ in range(RING)]
        + [pltpu.VMEM((CH, D), jnp.float32) for _ in range(RING)]
        + [pltpu.VMEM_SHARED((N_PAD, D), jnp.float32)]
        + [pltpu.SemaphoreType.DMA for _ in range(3 * RING)]
    )
    mesh = plsc.VectorSubcoreMesh(core_axis_name="c", subcore_axis_name="s")

    @functools.partial(
        pl.kernel,
        out_type=jax.ShapeDtypeStruct((NC, N, D), jnp.float32),
        mesh=mesh,
        scratch_types=scratch,
        interpret=interpret,
    )
    def sc_stage(emb_hbm, dst_hbm, z_hbm, out_hbm, *rest):
        dsti = rest[:RING]
        ebuf = rest[RING:2 * RING]
        agg_sh = rest[2 * RING]
        dsem = rest[2 * RING + 1:3 * RING + 1]
        esem = rest[3 * RING + 1:4 * RING + 1]
        ssem = rest[4 * RING + 1:]

        cid = lax.axis_index("c")
        sid = lax.axis_index("s")
        wid = cid * NS + sid
        base = wid * EPW

        def load(j, u):
            eb = base + j * CH
            pltpu.async_copy(dst_hbm.at[pl.ds(eb, CH)], dsti[u], dsem[u])
            pltpu.async_copy(emb_hbm.at[pl.ds(eb, CH)], ebuf[u], esem[u])

        def wait_load(u):
            pltpu.make_async_copy(
                dst_hbm.at[pl.ds(0, CH)], dsti[u], dsem[u]).wait()
            pltpu.make_async_copy(
                emb_hbm.at[pl.ds(0, CH)], ebuf[u], esem[u]).wait()

        def scat(u):
            pltpu.async_copy(ebuf[u], agg_sh.at[dsti[u]], ssem[u], add=True)

        def wait_scat(u):
            pltpu.make_async_copy(
                ebuf[u], agg_sh.at[dsti[u]], ssem[u]).wait()

        pltpu.sync_copy(z_hbm, agg_sh.at[pl.ds(sid * ZROWS, ZROWS)])
        plsc.subcore_barrier()

        for j in range(AHEAD):
            load(j, j)
        for j in range(RING):
            wait_load(j)
            scat(j)
            if j >= AHEAD:
                wait_scat(j - AHEAD)
            load(j + AHEAD, (j + AHEAD) % RING)

        loop_end = ((NCH - AHEAD) // RING) * RING

        @pl.loop(RING, loop_end, step=RING)
        def _(jb):
            for u in range(RING):
                j = jb + u
                wait_load(u)
                scat(u)
                wait_scat((u + AHEAD) % RING)
                load(j + AHEAD, (u + AHEAD) % RING)

        for j in range(loop_end, NCH):
            u = j % RING
            wait_load(u)
            scat(u)
            wait_scat((u + AHEAD) % RING)
            if j + AHEAD < NCH:
                load(j + AHEAD, (j + AHEAD) % RING)
        for j in range(NCH - AHEAD, NCH):
            wait_scat(j % RING)

        plsc.subcore_barrier()
        pltpu.sync_copy(
            agg_sh.at[pl.ds(sid * OROWS, OROWS)],
            out_hbm.at[cid, pl.ds(sid * OROWS, OROWS)],
        )

        @pl.when(sid == 0)
        def _():
            pltpu.sync_copy(
                agg_sh.at[pl.ds(NS * OROWS, OTAIL)],
                out_hbm.at[cid, pl.ds(NS * OROWS, OTAIL)],
            )

    return sc_stage


def _make_sc_stage(with_gather, interpret=False):
    scratch = [
        pltpu.VMEM((CH,), jnp.int32),
        pltpu.VMEM((CH,), jnp.int32),
        pltpu.VMEM((CH, D), jnp.float32),
        pltpu.VMEM((CH, D), jnp.float32),
        pltpu.VMEM_SHARED((N_PAD, D), jnp.float32),
        pltpu.SemaphoreType.DMA,
        pltpu.SemaphoreType.DMA,
        pltpu.SemaphoreType.DMA,
        pltpu.SemaphoreType.DMA,
    ]
    if with_gather:
        scratch += [
            pltpu.VMEM((CH,), jnp.int32),
            pltpu.VMEM((CH,), jnp.int32),
            pltpu.VMEM((CH, D), jnp.float32),
            pltpu.VMEM((CH, D), jnp.float32),
            pltpu.SemaphoreType.DMA,
            pltpu.SemaphoreType.DMA,
            pltpu.SemaphoreType.DMA,
            pltpu.SemaphoreType.DMA,
        ]
    mesh = plsc.VectorSubcoreMesh(core_axis_name="c", subcore_axis_name="s")

    @functools.partial(
        pl.kernel,
        out_type=jax.ShapeDtypeStruct((NC, N, D), jnp.float32),
        mesh=mesh,
        scratch_types=scratch,
        interpret=interpret,
    )
    def sc_stage(*args):
        if with_gather:
            (emb_hbm, dst_hbm, z_hbm, src_hbm, h_hbm, out_hbm,
             dsti0, dsti1, ebuf0, ebuf1, agg_sh, sd0, sd1, se0, se1,
             srci0, srci1, rows0, rows1, ss0, ss1, sg0, sg1) = args
            srci = (srci0, srci1)
            rows = (rows0, rows1)
            ssem = (ss0, ss1)
            gsem = (sg0, sg1)
        else:
            (emb_hbm, dst_hbm, z_hbm, out_hbm,
             dsti0, dsti1, ebuf0, ebuf1, agg_sh, sd0, sd1, se0, se1) = args
        dsti = (dsti0, dsti1)
        ebuf = (ebuf0, ebuf1)
        dsem = (sd0, sd1)
        esem = (se0, se1)

        cid = lax.axis_index("c")
        sid = lax.axis_index("s")
        wid = cid * NS + sid
        base = wid * EPW

        def start_loads(j, s):
            eb = base + j * CH
            pltpu.make_async_copy(
                dst_hbm.at[pl.ds(eb, CH)], dsti[s], dsem[s]).start()
            pltpu.make_async_copy(
                emb_hbm.at[pl.ds(eb, CH)], ebuf[s], esem[s]).start()
            if with_gather:
                pltpu.make_async_copy(
                    src_hbm.at[pl.ds(eb, CH)], srci[s], ssem[s]).start()

        def wait_loads(s):
            pltpu.make_async_copy(
                dst_hbm.at[pl.ds(0, CH)], dsti[s], dsem[s]).wait()
            pltpu.make_async_copy(
                emb_hbm.at[pl.ds(0, CH)], ebuf[s], esem[s]).wait()

        def start_gather(s):
            pltpu.make_async_copy(
                src_hbm.at[pl.ds(0, CH)], srci[s], ssem[s]).wait()
            pltpu.make_async_copy(h_hbm.at[srci[s]], rows[s], gsem[s]).start()

        def process(s):
            wait_loads(s)
            if with_gather:
                pltpu.make_async_copy(
                    h_hbm.at[srci[s]], rows[s], gsem[s]).wait()

                @pl.loop(0, CH, step=2)
                def _(r):
                    for rr in range(2):
                        for q in range(D // 16):
                            sl = pl.ds(q * 16, 16)
                            v = (ebuf[s].at[r + rr, sl][...]
                                 + rows[s].at[r + rr, sl][...])
                            ebuf[s].at[r + rr, sl][...] = jnp.maximum(v, 0.0)

                pltpu.sync_copy(ebuf[s], agg_sh.at[dsti[s]], add=True)
            else:
                pltpu.sync_copy(ebuf[s], agg_sh.at[dsti[s]], add=True)

        pltpu.sync_copy(z_hbm, agg_sh.at[pl.ds(sid * ZROWS, ZROWS)])
        plsc.subcore_barrier()

        start_loads(0, 0)
        start_loads(1, 1)
        if with_gather:
            start_gather(0)

        @pl.loop(0, (NCH - 1) // 2)
        def _(k):
            j = 2 * k
            process(0)
            start_loads(j + 2, 0)
            if with_gather:
                start_gather(1)
            process(1)

            @pl.when(j + 3 < NCH)
            def _():
                start_loads(j + 3, 1)

            if with_gather:
                start_gather(0)

        process(0)

        plsc.subcore_barrier()
        pltpu.sync_copy(
            agg_sh.at[pl.ds(sid * OROWS, OROWS)],
            out_hbm.at[cid, pl.ds(sid * OROWS, OROWS)],
        )

        @pl.when(sid == 0)
        def _():
            pltpu.sync_copy(
                agg_sh.at[pl.ds(NS * OROWS, OTAIL)],
                out_hbm.at[cid, pl.ds(NS * OROWS, OTAIL)],
            )

    return sc_stage


def _node_body(h_ref, a_ref, s_ref, w1_ref, b1_ref, g1_ref, be1_ref,
               w2_ref, b2_ref, go_ref, bo_ref, o_ref, *, final_relu):
    agg = a_ref[0] + a_ref[1]
    z = s_ref[0, 0] * h_ref[...] + agg
    t = jnp.dot(z, w1_ref[...], preferred_element_type=jnp.float32)
    t = t + b1_ref[...]
    m = jnp.mean(t, axis=0, keepdims=True)
    v = jnp.mean((t - m) ** 2, axis=0, keepdims=True)
    t = (t - m) * lax.rsqrt(v + 1e-5) * g1_ref[...] + be1_ref[...]
    t = jnp.maximum(t, 0.0)
    u = jnp.dot(t, w2_ref[...], preferred_element_type=jnp.float32)
    u = u + b2_ref[...]
    m2 = jnp.mean(u, axis=0, keepdims=True)
    v2 = jnp.mean((u - m2) ** 2, axis=0, keepdims=True)
    y = (u - m2) * lax.rsqrt(v2 + 1e-5) * go_ref[...] + bo_ref[...]
    if final_relu:
        y = jnp.maximum(y, 0.0)
    o_ref[...] = y


def _node_stage(final_relu, interpret=False):
    return pl.pallas_call(
        functools.partial(_node_body, final_relu=final_relu),
        out_shape=jax.ShapeDtypeStruct((N, D), jnp.float32),
        interpret=interpret,
    )


def kernel(x, edge_index, edge_attr, node_table, edge_W, edge_b, eps,
           W1, b1, g1, be1, W2, b2, g_out, b_out):
    f32 = jnp.float32
    src = edge_index[0]
    dst = edge_index[1]
    c = node_table[0].astype(f32)

    bias0 = (edge_b[0] + c)[None].astype(f32)
    bias1 = edge_b[1][None].astype(f32)

    msg0 = _edge_stage(relu=True)(edge_attr, edge_W[0], bias0)
    e1 = _edge_stage(relu=False)(edge_attr, edge_W[1], bias1)

    zeros = jnp.zeros((ZROWS, D), f32)
    agg0 = _make_sc_scatter()(msg0, dst, zeros)

    s0 = (1.0 + eps[0]).astype(f32).reshape(1, 1)
    h1 = _node_stage(final_relu=True)(
        c[None], agg0, s0, W1[0], b1[0][None], g1[0][None], be1[0][None],
        W2[0], b2[0][None], g_out[0][None], b_out[0][None])

    agg1 = _make_sc_stage(with_gather=True)(e1, dst, zeros, src, h1)

    s1 = (1.0 + eps[1]).astype(f32).reshape(1, 1)
    h2 = _node_stage(final_relu=False)(
        h1, agg1, s1, W1[1], b1[1][None], g1[1][None], be1[1][None],
        W2[1], b2[1][None], g_out[1][None], b_out[1][None])
    return h2
